# baseline (speedup 1.0000x reference)
import jax
import jax.numpy as jnp
from jax.experimental import pallas as pl
from jax.experimental.pallas import tpu as pltpu

NUM_ADJS = 3
N = 4096
IN_F = 512
OUT_F = 512

CH = 256
NSTEPS = N // CH
NSLOT = 3


def _chunk_copy(a_hbm, a_buf, sems, step, slot):
    return pltpu.make_async_copy(
        a_hbm.at[:, pl.ds(step * CH, CH), :],
        a_buf.at[slot],
        sems.at[slot],
    )


def _fused_kernel(
    a_hbm, x_ref, w_ref, b_ref, o_ref, a_buf, xb_ref, wb_ref, ts_ref, sems
):
    step = pl.program_id(0)

    @pl.when(step == 0)
    def _prologue():
        for j in range(NSLOT):
            _chunk_copy(a_hbm, a_buf, sems, j, j).start()
        xb_ref[...] = x_ref[...].astype(jnp.bfloat16)
        wb_ref[...] = (w_ref[...] * (1.0 / NUM_ADJS)).astype(jnp.bfloat16)

    slot = jax.lax.rem(step, NSLOT)
    _chunk_copy(a_hbm, a_buf, sems, step, slot).wait()

    half = jax.lax.rem(step, 2)
    for i in range(NUM_ADJS):
        t = jnp.dot(
            a_buf[slot, i].astype(jnp.bfloat16),
            xb_ref[...],
            preferred_element_type=jnp.float32,
        )
        ts_ref[i, pl.ds(half * CH, CH), :] = t.astype(jnp.bfloat16)

    @pl.when(half == 1)
    def _project():
        acc = jnp.broadcast_to(b_ref[...], (2 * CH, OUT_F)).astype(jnp.float32)
        for i in range(NUM_ADJS):
            acc = acc + jnp.dot(
                ts_ref[i], wb_ref[i], preferred_element_type=jnp.float32
            )
        o_ref[...] = acc

    @pl.when(step + NSLOT < NSTEPS)
    def _refill():
        _chunk_copy(a_hbm, a_buf, sems, step + NSLOT, slot).start()


@jax.jit
def kernel(input_, adjs, adj_weight, bias):
    bias2d = bias.reshape(1, OUT_F)
    out = pl.pallas_call(
        _fused_kernel,
        grid=(NSTEPS,),
        in_specs=[
            pl.BlockSpec(memory_space=pl.ANY),
            pl.BlockSpec((N, IN_F), lambda m: (0, 0)),
            pl.BlockSpec((NUM_ADJS, IN_F, OUT_F), lambda m: (0, 0, 0)),
            pl.BlockSpec((1, OUT_F), lambda m: (0, 0)),
        ],
        out_specs=pl.BlockSpec((2 * CH, OUT_F), lambda m: (m // 2, 0)),
        out_shape=jax.ShapeDtypeStruct((N, OUT_F), jnp.float32),
        scratch_shapes=[
            pltpu.VMEM((NSLOT, NUM_ADJS, CH, N), jnp.float32),
            pltpu.VMEM((N, IN_F), jnp.bfloat16),
            pltpu.VMEM((NUM_ADJS, IN_F, OUT_F), jnp.bfloat16),
            pltpu.VMEM((NUM_ADJS, 2 * CH, OUT_F), jnp.bfloat16),
            pltpu.SemaphoreType.DMA((NSLOT,)),
        ],
    )(adjs, input_, adj_weight, bias2d)
    return out
